# Initial kernel scaffold; baseline (speedup 1.0000x reference)
import jax
import jax.numpy as jnp
from jax.experimental import pallas as pl


def kernel(x, W_embed, b_embed, W0, b0, W1, b1, W_cls, b_cls, edge_index):
    raise NotImplementedError("write your pallas kernel here")


# speedup vs baseline: 363.0861x; 363.0861x over previous
import functools

import jax
import jax.numpy as jnp
from jax import lax
from jax.experimental import pallas as pl
from jax.experimental.pallas import tpu as pltpu

N = 10000
NPAD = 10240
H = 64
AUG = 128

f32 = jnp.float32
bf16 = jnp.bfloat16


def _cast_body(b_ref, o_ref):
    o_ref[...] = b_ref[...].astype(bf16)


def _cast_to_bf16(b2d):
    CB = 256
    return pl.pallas_call(
        _cast_body,
        grid=(NPAD // CB,),
        in_specs=[pl.BlockSpec((CB, NPAD), lambda i: (i, 0))],
        out_specs=pl.BlockSpec((CB, NPAD), lambda i: (i, 0)),
        out_shape=jax.ShapeDtypeStruct((NPAD, NPAD), bf16),
        compiler_params=pltpu.CompilerParams(
            dimension_semantics=("arbitrary",)),
    )(b2d)


def _mask2_body(bl_ref, br_ref, be_ref, o_ref, acc_ref, *, bm, bn, nk):
    k = pl.program_id(2)

    @pl.when(k == 0)
    def _():
        acc_ref[...] = jnp.zeros_like(acc_ref)

    acc_ref[...] += jnp.dot(bl_ref[...], br_ref[...],
                            preferred_element_type=f32)

    @pl.when(k == nk - 1)
    def _():
        i = pl.program_id(0)
        j = pl.program_id(1)
        c = acc_ref[...]
        m = (c > 0.5) & (be_ref[...] < bf16(0.5))
        rows = i * bm + lax.broadcasted_iota(jnp.int32, (bm, bn), 0)
        cols = j * bn + lax.broadcasted_iota(jnp.int32, (bm, bn), 1)
        m = m & (rows != cols)
        o_ref[...] = m.astype(bf16)


def _mask2(b_bf):
    BM, BN, BK = 1024, 1024, 2048
    nk = NPAD // BK
    return pl.pallas_call(
        functools.partial(_mask2_body, bm=BM, bn=BN, nk=nk),
        grid=(NPAD // BM, NPAD // BN, nk),
        in_specs=[
            pl.BlockSpec((BM, BK), lambda i, j, k: (i, k)),
            pl.BlockSpec((BK, BN), lambda i, j, k: (k, j)),
            pl.BlockSpec((BM, BN), lambda i, j, k: (i, j)),
        ],
        out_specs=pl.BlockSpec((BM, BN), lambda i, j, k: (i, j)),
        out_shape=jax.ShapeDtypeStruct((NPAD, NPAD), bf16),
        scratch_shapes=[pltpu.VMEM((BM, BN), f32)],
        compiler_params=pltpu.CompilerParams(
            dimension_semantics=("parallel", "parallel", "arbitrary")),
    )(b_bf, b_bf, b_bf)


def _embed_body(x_ref, w_ref, b_ref, o_ref, *, bm):
    h = jnp.dot(x_ref[...], w_ref[...], preferred_element_type=f32)
    h = jnp.maximum(h + b_ref[...], 0.0)
    ones = jnp.ones((bm, 1), f32)
    zeros = jnp.zeros((bm, AUG - H - 1), f32)
    o_ref[...] = jnp.concatenate([h, ones, zeros], axis=1)


def _embed(xp, w_embed, b_embed):
    BM = 1024
    return pl.pallas_call(
        functools.partial(_embed_body, bm=BM),
        grid=(NPAD // BM,),
        in_specs=[
            pl.BlockSpec((BM, 128), lambda i: (i, 0)),
            pl.BlockSpec((128, H), lambda i: (0, 0)),
            pl.BlockSpec((1, H), lambda i: (0, 0)),
        ],
        out_specs=pl.BlockSpec((BM, AUG), lambda i: (i, 0)),
        out_shape=jax.ShapeDtypeStruct((NPAD, AUG), f32),
        compiler_params=pltpu.CompilerParams(
            dimension_semantics=("arbitrary",)),
    )(xp, w_embed, b_embed)


def _layer_body(b_ref, m_ref, r_ref, ri_ref, wr_ref, w1_ref, w2_ref,
                bias_ref, o_ref, acc1_ref, acc2_ref, *, bm, nk):
    k = pl.program_id(1)

    @pl.when(k == 0)
    def _():
        acc1_ref[...] = jnp.zeros_like(acc1_ref)
        acc2_ref[...] = jnp.zeros_like(acc2_ref)

    r = r_ref[...]
    acc1_ref[...] += jnp.dot(b_ref[...].astype(f32), r,
                             preferred_element_type=f32)
    acc2_ref[...] += jnp.dot(m_ref[...].astype(f32), r,
                             preferred_element_type=f32)

    @pl.when(k == nk - 1)
    def _():
        a1 = acc1_ref[...]
        a2 = acc2_ref[...]
        n1 = a1[:, :H] / jnp.maximum(a1[:, H:H + 1], 1.0)
        n2 = a2[:, :H] / jnp.maximum(a2[:, H:H + 1], 1.0)
        rloc = ri_ref[:, :H]
        h = (jnp.dot(rloc, wr_ref[...], preferred_element_type=f32)
             + jnp.dot(n1, w1_ref[...], preferred_element_type=f32)
             + jnp.dot(n2, w2_ref[...], preferred_element_type=f32)
             + bias_ref[...])
        h = jnp.maximum(h, 0.0)
        ones = jnp.ones((bm, 1), f32)
        zeros = jnp.zeros((bm, AUG - H - 1), f32)
        o_ref[...] = jnp.concatenate([h, ones, zeros], axis=1)


def _layer(b_bf, m2_bf, raug, w_r, w_n1, w_n2, bias):
    BM, BK = 1024, 2048
    nk = NPAD // BK
    return pl.pallas_call(
        functools.partial(_layer_body, bm=BM, nk=nk),
        grid=(NPAD // BM, nk),
        in_specs=[
            pl.BlockSpec((BM, BK), lambda i, k: (i, k)),
            pl.BlockSpec((BM, BK), lambda i, k: (i, k)),
            pl.BlockSpec((BK, AUG), lambda i, k: (k, 0)),
            pl.BlockSpec((BM, AUG), lambda i, k: (i, 0)),
            pl.BlockSpec((H, H), lambda i, k: (0, 0)),
            pl.BlockSpec((H, H), lambda i, k: (0, 0)),
            pl.BlockSpec((H, H), lambda i, k: (0, 0)),
            pl.BlockSpec((1, H), lambda i, k: (0, 0)),
        ],
        out_specs=pl.BlockSpec((BM, AUG), lambda i, k: (i, 0)),
        out_shape=jax.ShapeDtypeStruct((NPAD, AUG), f32),
        scratch_shapes=[pltpu.VMEM((BM, AUG), f32),
                        pltpu.VMEM((BM, AUG), f32)],
        compiler_params=pltpu.CompilerParams(
            dimension_semantics=("parallel", "arbitrary")),
    )(b_bf, m2_bf, raug, raug, w_r, w_n1, w_n2, bias)


def _cls_body(r0_ref, r1_ref, r2_ref, w0_ref, w1_ref, w2_ref, b_ref, o_ref):
    h = (jnp.dot(r0_ref[:, :H], w0_ref[...], preferred_element_type=f32)
         + jnp.dot(r1_ref[:, :H], w1_ref[...], preferred_element_type=f32)
         + jnp.dot(r2_ref[:, :H], w2_ref[...], preferred_element_type=f32)
         + b_ref[...])
    o_ref[...] = h


def _classifier(r0, r1, r2, wc0, wc1, wc2, b_cls, n_cls):
    BM = 1024
    return pl.pallas_call(
        _cls_body,
        grid=(NPAD // BM,),
        in_specs=[
            pl.BlockSpec((BM, AUG), lambda i: (i, 0)),
            pl.BlockSpec((BM, AUG), lambda i: (i, 0)),
            pl.BlockSpec((BM, AUG), lambda i: (i, 0)),
            pl.BlockSpec((H, n_cls), lambda i: (0, 0)),
            pl.BlockSpec((H, n_cls), lambda i: (0, 0)),
            pl.BlockSpec((H, n_cls), lambda i: (0, 0)),
            pl.BlockSpec((1, n_cls), lambda i: (0, 0)),
        ],
        out_specs=pl.BlockSpec((BM, n_cls), lambda i: (i, 0)),
        out_shape=jax.ShapeDtypeStruct((NPAD, n_cls), f32),
        compiler_params=pltpu.CompilerParams(
            dimension_semantics=("arbitrary",)),
    )(r0, r1, r2, wc0, wc1, wc2, b_cls)


def _build_b_dense(src, dst):
    ok = src != dst
    val = ok.astype(f32)
    b = jnp.zeros((NPAD, NPAD), f32)
    b = b.at[src, dst].max(val)
    b = b.at[dst, src].max(val)
    return b


def kernel(x, W_embed, b_embed, W0, b0, W1, b1, W_cls, b_cls, edge_index):
    src = edge_index[0].astype(jnp.int32)
    dst = edge_index[1].astype(jnp.int32)

    b2d = _build_b_dense(src, dst)
    b_bf = _cast_to_bf16(b2d)
    m2_bf = _mask2(b_bf)

    xp = jnp.pad(x, ((0, NPAD - N), (0, 0)))
    r0 = _embed(xp, W_embed, b_embed.reshape(1, H))

    def layer_weights(W, b):
        w_r = W[0:H] + W[2 * H:3 * H]
        return w_r, W[H:2 * H], W[3 * H:4 * H], b.reshape(1, H)

    r1 = _layer(b_bf, m2_bf, r0, *layer_weights(W0, b0))
    r2 = _layer(b_bf, m2_bf, r1, *layer_weights(W1, b1))

    n_cls = W_cls.shape[1]
    out = _classifier(r0, r1, r2, W_cls[0:H], W_cls[H:2 * H],
                      W_cls[2 * H:3 * H], b_cls.reshape(1, n_cls), n_cls)
    return out[:N]


# speedup vs baseline: 381.3582x; 1.0503x over previous
import functools

import jax
import jax.numpy as jnp
from jax import lax
from jax.experimental import pallas as pl
from jax.experimental.pallas import tpu as pltpu
from jax.experimental.pallas import tpu_sc as plsc

N = 10000
NPAD = 10240
H = 64
AUG = 128

f32 = jnp.float32
bf16 = jnp.bfloat16


def _cast_body(b_ref, o_ref):
    o_ref[...] = b_ref[...].astype(bf16)


def _cast_to_bf16(b2d):
    CB = 256
    return pl.pallas_call(
        _cast_body,
        grid=(NPAD // CB,),
        in_specs=[pl.BlockSpec((CB, NPAD), lambda i: (i, 0))],
        out_specs=pl.BlockSpec((CB, NPAD), lambda i: (i, 0)),
        out_shape=jax.ShapeDtypeStruct((NPAD, NPAD), bf16),
        compiler_params=pltpu.CompilerParams(
            dimension_semantics=("arbitrary",)),
    )(b2d)


def _mask2_body(bl_ref, br_ref, be_ref, o_ref, acc_ref, *, bm, bn, nk):
    k = pl.program_id(2)

    @pl.when(k == 0)
    def _():
        acc_ref[...] = jnp.zeros_like(acc_ref)

    acc_ref[...] += jnp.dot(bl_ref[...], br_ref[...],
                            preferred_element_type=f32)

    @pl.when(k == nk - 1)
    def _():
        i = pl.program_id(0)
        j = pl.program_id(1)
        c = acc_ref[...]
        m = (c > 0.5) & (be_ref[...] < bf16(0.5))
        rows = i * bm + lax.broadcasted_iota(jnp.int32, (bm, bn), 0)
        cols = j * bn + lax.broadcasted_iota(jnp.int32, (bm, bn), 1)
        m = m & (rows != cols)
        o_ref[...] = m.astype(bf16)


def _mask2(b_bf):
    BM, BN, BK = 1024, 1024, 2048
    nk = NPAD // BK
    return pl.pallas_call(
        functools.partial(_mask2_body, bm=BM, bn=BN, nk=nk),
        grid=(NPAD // BM, NPAD // BN, nk),
        in_specs=[
            pl.BlockSpec((BM, BK), lambda i, j, k: (i, k)),
            pl.BlockSpec((BK, BN), lambda i, j, k: (k, j)),
            pl.BlockSpec((BM, BN), lambda i, j, k: (i, j)),
        ],
        out_specs=pl.BlockSpec((BM, BN), lambda i, j, k: (i, j)),
        out_shape=jax.ShapeDtypeStruct((NPAD, NPAD), bf16),
        scratch_shapes=[pltpu.VMEM((BM, BN), f32)],
        compiler_params=pltpu.CompilerParams(
            dimension_semantics=("parallel", "parallel", "arbitrary")),
    )(b_bf, b_bf, b_bf)


def _embed_body(x_ref, w_ref, b_ref, o_ref, *, bm):
    h = jnp.dot(x_ref[...], w_ref[...], preferred_element_type=f32)
    h = jnp.maximum(h + b_ref[...], 0.0)
    ones = jnp.ones((bm, 1), f32)
    zeros = jnp.zeros((bm, AUG - H - 1), f32)
    o_ref[...] = jnp.concatenate([h, ones, zeros], axis=1)


def _embed(xp, w_embed, b_embed):
    BM = 1024
    return pl.pallas_call(
        functools.partial(_embed_body, bm=BM),
        grid=(NPAD // BM,),
        in_specs=[
            pl.BlockSpec((BM, 128), lambda i: (i, 0)),
            pl.BlockSpec((128, H), lambda i: (0, 0)),
            pl.BlockSpec((1, H), lambda i: (0, 0)),
        ],
        out_specs=pl.BlockSpec((BM, AUG), lambda i: (i, 0)),
        out_shape=jax.ShapeDtypeStruct((NPAD, AUG), f32),
        compiler_params=pltpu.CompilerParams(
            dimension_semantics=("arbitrary",)),
    )(xp, w_embed, b_embed)


def _layer_body(b_ref, m_ref, r_ref, ri_ref, wr_ref, w1_ref, w2_ref,
                bias_ref, o_ref, acc1_ref, acc2_ref, *, bm, nk):
    k = pl.program_id(1)

    @pl.when(k == 0)
    def _():
        acc1_ref[...] = jnp.zeros_like(acc1_ref)
        acc2_ref[...] = jnp.zeros_like(acc2_ref)

    r = r_ref[...]
    acc1_ref[...] += jnp.dot(b_ref[...].astype(f32), r,
                             preferred_element_type=f32)
    acc2_ref[...] += jnp.dot(m_ref[...].astype(f32), r,
                             preferred_element_type=f32)

    @pl.when(k == nk - 1)
    def _():
        a1 = acc1_ref[...]
        a2 = acc2_ref[...]
        n1 = a1[:, :H] / jnp.maximum(a1[:, H:H + 1], 1.0)
        n2 = a2[:, :H] / jnp.maximum(a2[:, H:H + 1], 1.0)
        rloc = ri_ref[:, :H]
        h = (jnp.dot(rloc, wr_ref[...], preferred_element_type=f32)
             + jnp.dot(n1, w1_ref[...], preferred_element_type=f32)
             + jnp.dot(n2, w2_ref[...], preferred_element_type=f32)
             + bias_ref[...])
        h = jnp.maximum(h, 0.0)
        ones = jnp.ones((bm, 1), f32)
        zeros = jnp.zeros((bm, AUG - H - 1), f32)
        o_ref[...] = jnp.concatenate([h, ones, zeros], axis=1)


def _layer(b_bf, m2_bf, raug, w_r, w_n1, w_n2, bias):
    BM, BK = 1024, 2048
    nk = NPAD // BK
    return pl.pallas_call(
        functools.partial(_layer_body, bm=BM, nk=nk),
        grid=(NPAD // BM, nk),
        in_specs=[
            pl.BlockSpec((BM, BK), lambda i, k: (i, k)),
            pl.BlockSpec((BM, BK), lambda i, k: (i, k)),
            pl.BlockSpec((BK, AUG), lambda i, k: (k, 0)),
            pl.BlockSpec((BM, AUG), lambda i, k: (i, 0)),
            pl.BlockSpec((H, H), lambda i, k: (0, 0)),
            pl.BlockSpec((H, H), lambda i, k: (0, 0)),
            pl.BlockSpec((H, H), lambda i, k: (0, 0)),
            pl.BlockSpec((1, H), lambda i, k: (0, 0)),
        ],
        out_specs=pl.BlockSpec((BM, AUG), lambda i, k: (i, 0)),
        out_shape=jax.ShapeDtypeStruct((NPAD, AUG), f32),
        scratch_shapes=[pltpu.VMEM((BM, AUG), f32),
                        pltpu.VMEM((BM, AUG), f32)],
        compiler_params=pltpu.CompilerParams(
            dimension_semantics=("parallel", "arbitrary")),
    )(b_bf, m2_bf, raug, raug, w_r, w_n1, w_n2, bias)


def _cls_body(r0_ref, r1_ref, r2_ref, w0_ref, w1_ref, w2_ref, b_ref, o_ref):
    h = (jnp.dot(r0_ref[:, :H], w0_ref[...], preferred_element_type=f32)
         + jnp.dot(r1_ref[:, :H], w1_ref[...], preferred_element_type=f32)
         + jnp.dot(r2_ref[:, :H], w2_ref[...], preferred_element_type=f32)
         + b_ref[...])
    o_ref[...] = h


def _classifier(r0, r1, r2, wc0, wc1, wc2, b_cls, n_cls):
    BM = 1024
    return pl.pallas_call(
        _cls_body,
        grid=(NPAD // BM,),
        in_specs=[
            pl.BlockSpec((BM, AUG), lambda i: (i, 0)),
            pl.BlockSpec((BM, AUG), lambda i: (i, 0)),
            pl.BlockSpec((BM, AUG), lambda i: (i, 0)),
            pl.BlockSpec((H, n_cls), lambda i: (0, 0)),
            pl.BlockSpec((H, n_cls), lambda i: (0, 0)),
            pl.BlockSpec((H, n_cls), lambda i: (0, 0)),
            pl.BlockSpec((1, n_cls), lambda i: (0, 0)),
        ],
        out_specs=pl.BlockSpec((BM, n_cls), lambda i: (i, 0)),
        out_shape=jax.ShapeDtypeStruct((NPAD, n_cls), f32),
        compiler_params=pltpu.CompilerParams(
            dimension_semantics=("arbitrary",)),
    )(r0, r1, r2, wc0, wc1, wc2, b_cls)


_NW = 32
_CE = 8000
_ZCH = 32768
_CAP = 24608
_WIN = 8


def _sc_body(src_hbm, dst_hbm, b_hbm, zbuf, sbuf, dbuf, idx_a, idx_b,
             ones16, sem_z, sem_s):
    rows = NPAD // _NW
    nz = rows * NPAD // _ZCH
    e_total = src_hbm.shape[0]
    nch = e_total // _CE
    i32 = jnp.int32

    wid = lax.axis_index("s") * 2 + lax.axis_index("c")
    lo = wid * rows
    hi = lo + rows
    base = lo * NPAD
    safe = base + NPAD - 16
    safe_vec = jnp.full((16,), safe, i32)

    def zb(i, _):
        zbuf[pl.ds(i * 16, 16)] = jnp.zeros((16,), f32)
        return 0
    lax.fori_loop(0, _ZCH // 16, zb, 0)
    ones16[...] = jnp.ones((16,), f32)

    def fire_z(i, _):
        pltpu.async_copy(zbuf, b_hbm.at[pl.ds(base + i * _ZCH, _ZCH)], sem_z)
        return 0
    lax.fori_loop(0, nz, fire_z, 0)

    trash = _CAP - 16
    lov = jnp.full((16,), 1, i32) * lo
    hiv = lov + rows

    def scan_body(i, carry):
        ca, cb = carry
        s = sbuf[pl.ds(i * 16, 16)]
        d = dbuf[pl.ds(i * 16, 16)]
        ns = s != d
        ma = ns & (s >= lov) & (s < hiv)
        mb = ns & (d >= lov) & (d < hiv)
        ia = s * NPAD + d
        ib = d * NPAD + s
        pa = plsc.cumsum(ma.astype(i32))
        pb = plsc.cumsum(mb.astype(i32))
        plsc.store_scatter(idx_a, [jnp.where(ma, ca + pa - 1, trash)], ia)
        plsc.store_scatter(idx_b, [jnp.where(mb, cb + pb - 1, trash)], ib)
        ca = ca + jnp.sum(ma.astype(i32))
        cb = cb + jnp.sum(mb.astype(i32))
        return ca, cb

    ca = jnp.int32(0)
    cb = jnp.int32(0)
    for c in range(nch):
        pltpu.sync_copy(src_hbm.at[pl.ds(c * _CE, _CE)], sbuf)
        pltpu.sync_copy(dst_hbm.at[pl.ds(c * _CE, _CE)], dbuf)
        ca, cb = lax.fori_loop(0, _CE // 16, scan_body, (ca, cb))

    idx_a[pl.ds(ca, 16)] = safe_vec
    idx_b[pl.ds(cb, 16)] = safe_vec

    def drain_z(i, _):
        pltpu.make_async_copy(
            zbuf, b_hbm.at[pl.ds(base, _ZCH)], sem_z).wait()
        return 0
    lax.fori_loop(0, nz, drain_z, 0)

    def wait_one():
        pltpu.make_async_copy(ones16, b_hbm.at[safe_vec], sem_s).wait()

    def scatter(idx_ref, cnt):
        ng = (cnt + 15) // 16

        def body(j, _):
            @pl.when(j >= _WIN)
            def _():
                wait_one()
            v = idx_ref[pl.ds(j * 16, 16)]
            pltpu.async_copy(ones16, b_hbm.at[v], sem_s)
            return 0
        lax.fori_loop(0, ng, body, 0)

        def drain(j, _):
            wait_one()
            return 0
        lax.fori_loop(0, jnp.minimum(ng, _WIN), drain, 0)

    scatter(idx_a, ca)
    scatter(idx_b, cb)

    pltpu.sync_copy(zbuf.at[pl.ds(0, 16)], b_hbm.at[pl.ds(safe, 16)])


def _build_b_flat_sc(src, dst):
    mesh = plsc.VectorSubcoreMesh(core_axis_name="c", subcore_axis_name="s",
                                  num_cores=2, num_subcores=16)
    fn = pl.kernel(
        _sc_body,
        out_type=jax.ShapeDtypeStruct((NPAD * NPAD,), f32),
        mesh=mesh,
        compiler_params=pltpu.CompilerParams(needs_layout_passes=False),
        scratch_types=[
            pltpu.VMEM((_ZCH,), f32),
            pltpu.VMEM((_CE,), jnp.int32),
            pltpu.VMEM((_CE,), jnp.int32),
            pltpu.VMEM((_CAP,), jnp.int32),
            pltpu.VMEM((_CAP,), jnp.int32),
            pltpu.VMEM((16,), f32),
            pltpu.SemaphoreType.DMA,
            pltpu.SemaphoreType.DMA,
        ],
    )
    return fn(src, dst)


def _build_b_dense(src, dst):
    return _build_b_flat_sc(src, dst).reshape(NPAD, NPAD)


def kernel(x, W_embed, b_embed, W0, b0, W1, b1, W_cls, b_cls, edge_index):
    src = edge_index[0].astype(jnp.int32)
    dst = edge_index[1].astype(jnp.int32)

    b2d = _build_b_dense(src, dst)
    b_bf = _cast_to_bf16(b2d)
    m2_bf = _mask2(b_bf)

    xp = jnp.pad(x, ((0, NPAD - N), (0, 0)))
    r0 = _embed(xp, W_embed, b_embed.reshape(1, H))

    def layer_weights(W, b):
        w_r = W[0:H] + W[2 * H:3 * H]
        return w_r, W[H:2 * H], W[3 * H:4 * H], b.reshape(1, H)

    r1 = _layer(b_bf, m2_bf, r0, *layer_weights(W0, b0))
    r2 = _layer(b_bf, m2_bf, r1, *layer_weights(W1, b1))

    n_cls = W_cls.shape[1]
    out = _classifier(r0, r1, r2, W_cls[0:H], W_cls[H:2 * H],
                      W_cls[2 * H:3 * H], b_cls.reshape(1, n_cls), n_cls)
    return out[:N]
